# Initial kernel scaffold; baseline (speedup 1.0000x reference)
import jax
import jax.numpy as jnp
from jax import lax
from jax.experimental import pallas as pl
from jax.experimental.pallas import tpu as pltpu
from jax.experimental.pallas import tpu_sc as plsc

_B = 16384
_XW = 16
_DIMS = (3, 4, 6, 4, 50)
_OUT_W = _XW + sum(_DIMS)
_NC, _NS = 2, 16
_NW = _NC * _NS
_BPW = _B // _NW
_CH = 128
_NCH = _BPW // _CH

_COL_OFF = []
_off = _XW
for _d in _DIMS:
    _COL_OFF.append(_off)
    _off += _d


def _body(x_hbm, ycols_hbm, e0, e1, e2, e3, e4, out_hbm, idx_v, stage_v):
    tables = (e0, e1, e2, e3, e4)
    wid = lax.axis_index("s") * _NC + lax.axis_index("c")
    base = wid * _BPW

    for t in range(5):
        for c in range(_NCH):
            pltpu.sync_copy(
                ycols_hbm.at[pl.ds(t * _B + base + c * _CH, _CH)],
                idx_v.at[t * _NCH + c],
            )

    pltpu.sync_copy(
        x_hbm.at[pl.ds(base, _BPW), :],
        stage_v.at[:, pl.ds(0, _XW)],
    )

    for t in range(5):
        d = _DIMS[t]
        off = _COL_OFF[t]
        for c in range(_NCH):
            pltpu.sync_copy(
                tables[t].at[idx_v.at[t * _NCH + c]],
                stage_v.at[pl.ds(c * _CH, _CH), pl.ds(off, d)],
            )

    pltpu.sync_copy(stage_v, out_hbm.at[pl.ds(base, _BPW), :])


def kernel(x, y, emb0, emb1, emb2, emb3, emb4):
    ycols = y.T.reshape(5 * _B)
    mesh = plsc.VectorSubcoreMesh(core_axis_name="c", subcore_axis_name="s")
    kern = pl.kernel(
        _body,
        out_type=jax.ShapeDtypeStruct((_B, _OUT_W), jnp.float32),
        mesh=mesh,
        scratch_types=[
            pltpu.VMEM((5 * _NCH, _CH), jnp.int32),
            pltpu.VMEM((_BPW, _OUT_W), jnp.float32),
        ],
    )
    return kern(x, ycols, emb0, emb1, emb2, emb3, emb4)


# speedup vs baseline: 4.7483x; 4.7483x over previous
import jax
import jax.numpy as jnp
from jax import lax
from jax.experimental import pallas as pl
from jax.experimental.pallas import tpu as pltpu
from jax.experimental.pallas import tpu_sc as plsc

_B = 16384
_XW = 16
_DIMS = (3, 4, 6, 4, 50)
_OUT_W = _XW + sum(_DIMS)

_NC, _NS = 2, 16
_NW = _NC * _NS
_BPW = _B // _NW
_NCHK = _BPW // 16

_TBASE = []
_a = 0
for _v, _d in zip((6, 7, 12, 7, 96), _DIMS):
    _TBASE.append(_a)
    _a += -(-(_v * _d) // 8) * 8
_TPACK = _a

_COL_OFF = []
_o = _XW
for _d in _DIMS:
    _COL_OFF.append(_o)
    _o += _d


def _body(xf_hbm, ycols_hbm, tbl_hbm, out_hbm, tbl_v, x_v, y_v, stage_v):
    wid = lax.axis_index("s") * _NC + lax.axis_index("c")
    base = wid * _BPW

    pltpu.sync_copy(tbl_hbm, tbl_v)
    pltpu.sync_copy(xf_hbm.at[pl.ds(base * _XW, _BPW * _XW)], x_v)
    for t in range(5):
        pltpu.sync_copy(
            ycols_hbm.at[pl.ds(t * _B + base, _BPW)],
            y_v.at[pl.ds(t * _BPW, _BPW)],
        )

    iota = jax.lax.iota(jnp.int32, 16)

    def chunk(c, carry):
        r0 = c * 16
        rows83 = iota * _OUT_W + r0 * _OUT_W

        for k in range(16):
            xvec = x_v[pl.ds((r0 + k) * _XW, 16)]
            plsc.store_scatter(stage_v, [iota + (r0 + k) * _OUT_W], xvec)

        for t in range(5):
            d = _DIMS[t]
            yt = y_v[pl.ds(t * _BPW + r0, 16)]
            srcb = yt * d + _TBASE[t]
            dstb = rows83 + _COL_OFF[t]
            for j in range(d):
                v = plsc.load_gather(tbl_v, [srcb + j])
                plsc.store_scatter(stage_v, [dstb + j], v)
        return carry

    lax.fori_loop(0, _NCHK, chunk, 0)

    pltpu.sync_copy(stage_v, out_hbm.at[pl.ds(base * _OUT_W, _BPW * _OUT_W)])


def kernel(x, y, emb0, emb1, emb2, emb3, emb4):
    ycols = y.T.reshape(5 * _B)
    xf = x.reshape(_B * _XW)
    pieces = []
    for e, b, nb in zip((emb0, emb1, emb2, emb3, emb4),
                        _TBASE, _TBASE[1:] + [_TPACK]):
        r = e.reshape(-1)
        pieces.append(r)
        pad = nb - b - r.shape[0]
        if pad:
            pieces.append(jnp.zeros((pad,), jnp.float32))
    tbl = jnp.concatenate(pieces)

    mesh = plsc.VectorSubcoreMesh(core_axis_name="c", subcore_axis_name="s")
    kern = pl.kernel(
        _body,
        out_type=jax.ShapeDtypeStruct((_B * _OUT_W,), jnp.float32),
        mesh=mesh,
        scratch_types=[
            pltpu.VMEM((_TPACK,), jnp.float32),
            pltpu.VMEM((_BPW * _XW,), jnp.float32),
            pltpu.VMEM((5 * _BPW,), jnp.int32),
            pltpu.VMEM((_BPW * _OUT_W,), jnp.float32),
        ],
        compiler_params=pltpu.CompilerParams(needs_layout_passes=False),
    )
    flat = kern(xf, ycols, tbl)
    return flat.reshape(_B, _OUT_W)
